# baseline (speedup 1.0000x reference)
import dataclasses
import functools

import jax
import jax.numpy as jnp
from jax import lax
from jax.experimental import pallas as pl
from jax.experimental.pallas import tpu as pltpu
from jax.experimental.pallas import tpu_sc as plsc

N_ENT = 1000000
N_REL = 1000
HALF_ENT = N_ENT // 2
HALF_REL = N_REL // 2
D = 64
B = 16384
MARGIN = 1.0

NC = 2
NS = 16
NW = NC * NS

R_PER_W = B // NW
CH = 128
C_PER_SUB = R_PER_W // CH

PACK_BLK = 5000
PACK_GRID = HALF_ENT // PACK_BLK

BLK = 2048
GRID = B // BLK


def _sc_compiler_params():
    cp = pltpu.CompilerParams()
    if "needs_layout_passes" in pltpu.CompilerParams.__dataclass_fields__:
        cp = dataclasses.replace(cp, needs_layout_passes=False)
    return cp


def _pack_body(x_ref, o_ref):
    j = pl.program_id(1)

    @pl.when(j == 0)
    def _():
        o_ref[:, :D] = x_ref[...]

    @pl.when(j == 1)
    def _():
        o_ref[:, D:] = x_ref[...]


def _pack_entity(ent):
    return pl.pallas_call(
        _pack_body,
        grid=(PACK_GRID, 2),
        in_specs=[pl.BlockSpec((PACK_BLK, D), lambda i, j: (i + j * PACK_GRID, 0))],
        out_specs=pl.BlockSpec((PACK_BLK, 2 * D), lambda i, j: (i, 0)),
        out_shape=jax.ShapeDtypeStruct((HALF_ENT, 2 * D), jnp.float32),
    )(ent)


def _gather_rows(ent2, rel2, eidx2d, ridx2d):
    mesh = plsc.VectorSubcoreMesh(core_axis_name="core", subcore_axis_name="subcore")

    row_type = jax.ShapeDtypeStruct((B, 2 * D), jnp.float32)

    @functools.partial(
        pl.kernel,
        out_type=(row_type, row_type, row_type, row_type, row_type),
        mesh=mesh,
        scratch_types=[
            pltpu.VMEM((CH,), jnp.int32),
            pltpu.VMEM((CH, 2 * D), jnp.float32),
            pltpu.SemaphoreType.DMA,
        ],
        compiler_params=_sc_compiler_params(),
    )
    def gk(ent_hbm, rel_hbm, eidx_hbm, ridx_hbm,
           h_out, t_out, hn_out, tn_out, rel_out, idx_v, rows_v, sem):
        wid = lax.axis_index("subcore") * NC + lax.axis_index("core")

        def chunk_gather(tab_hbm, idx_hbm, out_hbm, row, base):
            pltpu.sync_copy(idx_hbm.at[row], idx_v)
            pltpu.async_copy(tab_hbm.at[idx_v], rows_v, sem).wait()
            pltpu.sync_copy(rows_v, out_hbm.at[pl.ds(base, CH)])

        for k, out in enumerate((h_out, t_out, hn_out, tn_out)):
            @pl.loop(0, C_PER_SUB)
            def _(c, k=k, out=out):
                row = wid * C_PER_SUB + c
                chunk_gather(ent_hbm, eidx_hbm, out, k * (B // CH) + row,
                             row * CH)

        @pl.loop(0, C_PER_SUB)
        def _(c):
            row = wid * C_PER_SUB + c
            chunk_gather(rel_hbm, ridx_hbm, rel_out, row, row * CH)

    return gk(ent2, rel2, eidx2d, ridx2d)


def _loss_body(h_ref, t_ref, hn_ref, tn_ref, r_ref,
               ph_ref, pt_ref, phn_ref, ptn_ref, pr_ref, out_ref):
    i = pl.program_id(0)

    def pick(x2, p_ref):
        p = p_ref[...]
        return jnp.where(p != 0, x2[:, D:], x2[:, :D])

    def nrm(x):
        n = jnp.sqrt(jnp.sum(x * x, axis=1, keepdims=True))
        return x / (n + 1e-12)

    h = nrm(pick(h_ref[...], ph_ref))
    t = nrm(pick(t_ref[...], pt_ref))
    hn = nrm(pick(hn_ref[...], phn_ref))
    tn = nrm(pick(tn_ref[...], ptn_ref))
    r = pick(r_ref[...], pr_ref)
    pos = jnp.sum(jnp.abs(h + r - t), axis=1)
    neg = jnp.sum(jnp.abs(hn + r - tn), axis=1)
    part = jnp.sum(jnp.maximum(MARGIN + pos - neg, 0.0)) * (1.0 / B)

    @pl.when(i == 0)
    def _():
        out_ref[...] = jnp.zeros_like(out_ref)

    out_ref[...] += jnp.reshape(part, (1, 1))


def kernel(entity_embedding, relation_embedding, pos_h, pos_r, pos_t, neg_h, neg_t):
    ent2 = _pack_entity(entity_embedding)
    rel2 = jnp.reshape(relation_embedding, (HALF_REL, 2 * D))

    eidx = jnp.concatenate([pos_h, pos_t, neg_h, neg_t])
    epair = jnp.where(eidx >= HALF_ENT, eidx - HALF_ENT, eidx)
    epar = (eidx >= HALF_ENT).astype(jnp.int32)
    rpair = pos_r >> 1
    rpar = pos_r & 1

    g_h, g_t, g_hn, g_tn, g_rel = _gather_rows(
        ent2, rel2,
        epair.reshape(4 * B // CH, CH),
        rpair.reshape(B // CH, CH),
    )

    pars2d = jnp.concatenate([epar, rpar]).reshape(5 * B, 1)

    loss = pl.pallas_call(
        _loss_body,
        grid=(GRID,),
        in_specs=[
            pl.BlockSpec((BLK, 2 * D), lambda i: (i, 0)),
            pl.BlockSpec((BLK, 2 * D), lambda i: (i, 0)),
            pl.BlockSpec((BLK, 2 * D), lambda i: (i, 0)),
            pl.BlockSpec((BLK, 2 * D), lambda i: (i, 0)),
            pl.BlockSpec((BLK, 2 * D), lambda i: (i, 0)),
            pl.BlockSpec((BLK, 1), lambda i: (i, 0)),
            pl.BlockSpec((BLK, 1), lambda i: (i + GRID, 0)),
            pl.BlockSpec((BLK, 1), lambda i: (i + 2 * GRID, 0)),
            pl.BlockSpec((BLK, 1), lambda i: (i + 3 * GRID, 0)),
            pl.BlockSpec((BLK, 1), lambda i: (i + 4 * GRID, 0)),
        ],
        out_specs=pl.BlockSpec((1, 1), lambda i: (0, 0)),
        out_shape=jax.ShapeDtypeStruct((1, 1), jnp.float32),
    )(g_h, g_t, g_hn, g_tn, g_rel, pars2d, pars2d, pars2d, pars2d, pars2d)
    return loss[0, 0]


# speedup vs baseline: 1.0557x; 1.0557x over previous
import dataclasses
import functools

import jax
import jax.numpy as jnp
from jax import lax
from jax.experimental import pallas as pl
from jax.experimental.pallas import tpu as pltpu
from jax.experimental.pallas import tpu_sc as plsc

N_ENT = 1000000
N_REL = 1000
HALF_ENT = N_ENT // 2
HALF_REL = N_REL // 2
D = 64
B = 16384
MARGIN = 1.0

NC = 2
NS = 16
NW = NC * NS

R_PER_W = B // NW
CH = 128
C_PER_SUB = R_PER_W // CH

PACK_BLK = 5000
PACK_GRID = HALF_ENT // PACK_BLK

BLK = 2048
GRID = B // BLK


def _sc_compiler_params():
    cp = pltpu.CompilerParams()
    if "needs_layout_passes" in pltpu.CompilerParams.__dataclass_fields__:
        cp = dataclasses.replace(cp, needs_layout_passes=False)
    return cp


PACK_W = 6400
PACK_NW_FULL = N_ENT // PACK_W
PACK_TAIL = N_ENT - PACK_NW_FULL * PACK_W


def _pack_body(x_hbm, tail_hbm, o_hbm, xb, zb, sem, osem):
    def do_window(col0, w):
        pltpu.make_async_copy(
            x_hbm.at[:, pl.ds(col0, w)], xb.at[:, pl.ds(0, w)], sem
        ).start()
        pltpu.make_async_copy(
            x_hbm.at[:, pl.ds(col0, w)], xb.at[:, pl.ds(0, w)], sem
        ).wait()
        y = jnp.transpose(xb[:, pl.ds(0, w)])
        zb[pl.ds(0, w // 2), :D] = y[: w // 2]
        zb[pl.ds(0, w // 2), D:] = y[w // 2:]
        pltpu.make_async_copy(
            zb.at[pl.ds(0, w // 2), :], o_hbm.at[pl.ds(col0 // 2, w // 2)],
            osem,
        ).start()
        pltpu.make_async_copy(
            zb.at[pl.ds(0, w // 2), :], o_hbm.at[pl.ds(col0 // 2, w // 2)],
            osem,
        ).wait()

    @pl.loop(0, PACK_NW_FULL)
    def _(i):
        do_window(pl.multiple_of(i * PACK_W, 128), PACK_W)

    tcp = pltpu.make_async_copy(
        tail_hbm, o_hbm.at[pl.ds(PACK_NW_FULL * PACK_W // 2, PACK_TAIL // 2)],
        osem)
    tcp.start()
    tcp.wait()


def _pack_entity(ent):
    ent_t = jnp.transpose(ent)
    t = ent[PACK_NW_FULL * PACK_W:]
    tail = jnp.concatenate([t[: PACK_TAIL // 2], t[PACK_TAIL // 2:]], axis=1)
    return pl.pallas_call(
        _pack_body,
        in_specs=[
            pl.BlockSpec(memory_space=pltpu.MemorySpace.HBM),
            pl.BlockSpec(memory_space=pltpu.MemorySpace.HBM),
        ],
        out_specs=pl.BlockSpec(memory_space=pltpu.MemorySpace.HBM),
        out_shape=jax.ShapeDtypeStruct((HALF_ENT, 2 * D), jnp.float32),
        scratch_shapes=[
            pltpu.VMEM((D, PACK_W), jnp.float32),
            pltpu.VMEM((PACK_W // 2, 2 * D), jnp.float32),
            pltpu.SemaphoreType.DMA,
            pltpu.SemaphoreType.DMA,
        ],
    )(ent_t, tail)


def _gather_rows(ent2, rel2, eidx2d, ridx2d):
    mesh = plsc.VectorSubcoreMesh(core_axis_name="core", subcore_axis_name="subcore")

    row_type = jax.ShapeDtypeStruct((B, 2 * D), jnp.float32)

    @functools.partial(
        pl.kernel,
        out_type=(row_type, row_type, row_type, row_type, row_type),
        mesh=mesh,
        scratch_types=[
            pltpu.VMEM((CH,), jnp.int32),
            pltpu.VMEM((CH, 2 * D), jnp.float32),
            pltpu.SemaphoreType.DMA,
        ],
        compiler_params=_sc_compiler_params(),
    )
    def gk(ent_hbm, rel_hbm, eidx_hbm, ridx_hbm,
           h_out, t_out, hn_out, tn_out, rel_out, idx_v, rows_v, sem):
        wid = lax.axis_index("subcore") * NC + lax.axis_index("core")

        def chunk_gather(tab_hbm, idx_hbm, out_hbm, row, base):
            pltpu.sync_copy(idx_hbm.at[row], idx_v)
            pltpu.async_copy(tab_hbm.at[idx_v], rows_v, sem).wait()
            pltpu.sync_copy(rows_v, out_hbm.at[pl.ds(base, CH)])

        for k, out in enumerate((h_out, t_out, hn_out, tn_out)):
            @pl.loop(0, C_PER_SUB)
            def _(c, k=k, out=out):
                row = wid * C_PER_SUB + c
                chunk_gather(ent_hbm, eidx_hbm, out, k * (B // CH) + row,
                             row * CH)

        @pl.loop(0, C_PER_SUB)
        def _(c):
            row = wid * C_PER_SUB + c
            chunk_gather(rel_hbm, ridx_hbm, rel_out, row, row * CH)

    return gk(ent2, rel2, eidx2d, ridx2d)


def _loss_body(h_ref, t_ref, hn_ref, tn_ref, r_ref,
               ph_ref, pt_ref, phn_ref, ptn_ref, pr_ref, out_ref):
    i = pl.program_id(0)

    def pick(x2, p_ref):
        p = p_ref[...]
        return jnp.where(p != 0, x2[:, D:], x2[:, :D])

    def nrm(x):
        n = jnp.sqrt(jnp.sum(x * x, axis=1, keepdims=True))
        return x / (n + 1e-12)

    h = nrm(pick(h_ref[...], ph_ref))
    t = nrm(pick(t_ref[...], pt_ref))
    hn = nrm(pick(hn_ref[...], phn_ref))
    tn = nrm(pick(tn_ref[...], ptn_ref))
    r = pick(r_ref[...], pr_ref)
    pos = jnp.sum(jnp.abs(h + r - t), axis=1)
    neg = jnp.sum(jnp.abs(hn + r - tn), axis=1)
    part = jnp.sum(jnp.maximum(MARGIN + pos - neg, 0.0)) * (1.0 / B)

    @pl.when(i == 0)
    def _():
        out_ref[...] = jnp.zeros_like(out_ref)

    out_ref[...] += jnp.reshape(part, (1, 1))


def kernel(entity_embedding, relation_embedding, pos_h, pos_r, pos_t, neg_h, neg_t):
    ent2 = _pack_entity(entity_embedding)
    rel2 = jnp.reshape(relation_embedding, (HALF_REL, 2 * D))

    eidx = jnp.concatenate([pos_h, pos_t, neg_h, neg_t])
    blk = eidx // PACK_W
    rem = eidx % PACK_W
    hw = jnp.where(blk >= PACK_NW_FULL, PACK_TAIL // 2, PACK_W // 2)
    epair = jnp.minimum(blk, PACK_NW_FULL) * (PACK_W // 2) + rem % hw
    epar = rem // hw
    rpair = pos_r >> 1
    rpar = pos_r & 1

    g_h, g_t, g_hn, g_tn, g_rel = _gather_rows(
        ent2, rel2,
        epair.reshape(4 * B // CH, CH),
        rpair.reshape(B // CH, CH),
    )

    pars2d = jnp.concatenate([epar, rpar]).reshape(5 * B, 1)

    loss = pl.pallas_call(
        _loss_body,
        grid=(GRID,),
        in_specs=[
            pl.BlockSpec((BLK, 2 * D), lambda i: (i, 0)),
            pl.BlockSpec((BLK, 2 * D), lambda i: (i, 0)),
            pl.BlockSpec((BLK, 2 * D), lambda i: (i, 0)),
            pl.BlockSpec((BLK, 2 * D), lambda i: (i, 0)),
            pl.BlockSpec((BLK, 2 * D), lambda i: (i, 0)),
            pl.BlockSpec((BLK, 1), lambda i: (i, 0)),
            pl.BlockSpec((BLK, 1), lambda i: (i + GRID, 0)),
            pl.BlockSpec((BLK, 1), lambda i: (i + 2 * GRID, 0)),
            pl.BlockSpec((BLK, 1), lambda i: (i + 3 * GRID, 0)),
            pl.BlockSpec((BLK, 1), lambda i: (i + 4 * GRID, 0)),
        ],
        out_specs=pl.BlockSpec((1, 1), lambda i: (0, 0)),
        out_shape=jax.ShapeDtypeStruct((1, 1), jnp.float32),
    )(g_h, g_t, g_hn, g_tn, g_rel, pars2d, pars2d, pars2d, pars2d, pars2d)
    return loss[0, 0]


# speedup vs baseline: 1.9181x; 1.8168x over previous
import dataclasses
import functools

import jax
import jax.numpy as jnp
from jax import lax
from jax.experimental import pallas as pl
from jax.experimental.pallas import tpu as pltpu
from jax.experimental.pallas import tpu_sc as plsc

N_ENT = 1000000
N_REL = 1000
HALF_ENT = N_ENT // 2
HALF_REL = N_REL // 2
D = 64
B = 16384
MARGIN = 1.0

NC = 2
NS = 16
NW = NC * NS

R_PER_W = B // NW
CH = 128
C_PER_SUB = R_PER_W // CH

PACK_BLK = 5000
PACK_GRID = HALF_ENT // PACK_BLK

BLK = 2048
GRID = B // BLK


def _sc_compiler_params():
    cp = pltpu.CompilerParams()
    if "needs_layout_passes" in pltpu.CompilerParams.__dataclass_fields__:
        cp = dataclasses.replace(cp, needs_layout_passes=False)
    return cp


PACK_W = 6400
PACK_NW_FULL = N_ENT // PACK_W
PACK_TAIL = N_ENT - PACK_NW_FULL * PACK_W


def _pack_body(x_hbm, tail_hbm, o_hbm,
               xb0, xb1, zb0, zb1, isem0, isem1, osem0, osem1):
    xbs, zbs = (xb0, xb1), (zb0, zb1)
    isems, osems = (isem0, isem1), (osem0, osem1)
    W, HW, NWF = PACK_W, PACK_W // 2, PACK_NW_FULL

    def start_in(i, b):
        col0 = pl.multiple_of(i * W, 128)
        pltpu.make_async_copy(x_hbm.at[:, pl.ds(col0, W)], xbs[b], isems[b]).start()

    def wait_in(b):
        pltpu.make_async_copy(x_hbm.at[:, pl.ds(0, W)], xbs[b], isems[b]).wait()

    def compute(b):
        y = jnp.transpose(xbs[b][...])
        zbs[b][:, :D] = y[:HW]
        zbs[b][:, D:] = y[HW:]

    def start_out(i, b):
        pltpu.make_async_copy(
            zbs[b], o_hbm.at[pl.ds(pl.multiple_of(i * HW, 8), HW)], osems[b]
        ).start()

    def wait_out(b):
        pltpu.make_async_copy(zbs[b], o_hbm.at[pl.ds(0, HW)], osems[b]).wait()

    start_in(0, 0)

    @pl.loop(0, NWF // 2)
    def _(k):
        i0 = k * 2
        start_in(i0 + 1, 1)
        wait_in(0)

        @pl.when(k > 0)
        def _():
            wait_out(0)

        compute(0)
        start_out(i0, 0)

        @pl.when(k < NWF // 2 - 1)
        def _():
            start_in(i0 + 2, 0)

        wait_in(1)

        @pl.when(k > 0)
        def _():
            wait_out(1)

        compute(1)
        start_out(i0 + 1, 1)

    wait_out(0)
    wait_out(1)

    tcp = pltpu.make_async_copy(
        tail_hbm, o_hbm.at[pl.ds(NWF * W // 2, PACK_TAIL // 2)], osem0)
    tcp.start()
    tcp.wait()


def _pack_entity(ent):
    ent_t = jnp.transpose(ent)
    t = ent[PACK_NW_FULL * PACK_W:]
    tail = jnp.concatenate([t[: PACK_TAIL // 2], t[PACK_TAIL // 2:]], axis=1)
    return pl.pallas_call(
        _pack_body,
        in_specs=[
            pl.BlockSpec(memory_space=pltpu.MemorySpace.HBM),
            pl.BlockSpec(memory_space=pltpu.MemorySpace.HBM),
        ],
        out_specs=pl.BlockSpec(memory_space=pltpu.MemorySpace.HBM),
        out_shape=jax.ShapeDtypeStruct((HALF_ENT, 2 * D), jnp.float32),
        scratch_shapes=[
            pltpu.VMEM((D, PACK_W), jnp.float32),
            pltpu.VMEM((D, PACK_W), jnp.float32),
            pltpu.VMEM((PACK_W // 2, 2 * D), jnp.float32),
            pltpu.VMEM((PACK_W // 2, 2 * D), jnp.float32),
            pltpu.SemaphoreType.DMA,
            pltpu.SemaphoreType.DMA,
            pltpu.SemaphoreType.DMA,
            pltpu.SemaphoreType.DMA,
        ],
    )(ent_t, tail)


def _gather_rows(ent2, rel2, eidx2d, ridx2d):
    mesh = plsc.VectorSubcoreMesh(core_axis_name="core", subcore_axis_name="subcore")

    row_type = jax.ShapeDtypeStruct((B, 2 * D), jnp.float32)

    @functools.partial(
        pl.kernel,
        out_type=(row_type, row_type, row_type, row_type, row_type),
        mesh=mesh,
        scratch_types=[
            pltpu.VMEM((CH,), jnp.int32),
            pltpu.VMEM((CH, 2 * D), jnp.float32),
            pltpu.SemaphoreType.DMA,
        ],
        compiler_params=_sc_compiler_params(),
    )
    def gk(ent_hbm, rel_hbm, eidx_hbm, ridx_hbm,
           h_out, t_out, hn_out, tn_out, rel_out, idx_v, rows_v, sem):
        wid = lax.axis_index("subcore") * NC + lax.axis_index("core")

        def chunk_gather(tab_hbm, idx_hbm, out_hbm, row, base):
            pltpu.sync_copy(idx_hbm.at[row], idx_v)
            pltpu.async_copy(tab_hbm.at[idx_v], rows_v, sem).wait()
            pltpu.sync_copy(rows_v, out_hbm.at[pl.ds(base, CH)])

        for k, out in enumerate((h_out, t_out, hn_out, tn_out)):
            @pl.loop(0, C_PER_SUB)
            def _(c, k=k, out=out):
                row = wid * C_PER_SUB + c
                chunk_gather(ent_hbm, eidx_hbm, out, k * (B // CH) + row,
                             row * CH)

        @pl.loop(0, C_PER_SUB)
        def _(c):
            row = wid * C_PER_SUB + c
            chunk_gather(rel_hbm, ridx_hbm, rel_out, row, row * CH)

    return gk(ent2, rel2, eidx2d, ridx2d)


def _loss_body(h_ref, t_ref, hn_ref, tn_ref, r_ref,
               ph_ref, pt_ref, phn_ref, ptn_ref, pr_ref, out_ref):
    i = pl.program_id(0)

    def pick(x2, p_ref):
        p = p_ref[...]
        return jnp.where(p != 0, x2[:, D:], x2[:, :D])

    def nrm(x):
        n = jnp.sqrt(jnp.sum(x * x, axis=1, keepdims=True))
        return x / (n + 1e-12)

    h = nrm(pick(h_ref[...], ph_ref))
    t = nrm(pick(t_ref[...], pt_ref))
    hn = nrm(pick(hn_ref[...], phn_ref))
    tn = nrm(pick(tn_ref[...], ptn_ref))
    r = pick(r_ref[...], pr_ref)
    pos = jnp.sum(jnp.abs(h + r - t), axis=1)
    neg = jnp.sum(jnp.abs(hn + r - tn), axis=1)
    part = jnp.sum(jnp.maximum(MARGIN + pos - neg, 0.0)) * (1.0 / B)

    @pl.when(i == 0)
    def _():
        out_ref[...] = jnp.zeros_like(out_ref)

    out_ref[...] += jnp.reshape(part, (1, 1))


def kernel(entity_embedding, relation_embedding, pos_h, pos_r, pos_t, neg_h, neg_t):
    ent2 = _pack_entity(entity_embedding)
    rel2 = jnp.reshape(relation_embedding, (HALF_REL, 2 * D))

    eidx = jnp.concatenate([pos_h, pos_t, neg_h, neg_t])
    blk = eidx // PACK_W
    rem = eidx % PACK_W
    hw = jnp.where(blk >= PACK_NW_FULL, PACK_TAIL // 2, PACK_W // 2)
    epair = jnp.minimum(blk, PACK_NW_FULL) * (PACK_W // 2) + rem % hw
    epar = rem // hw
    rpair = pos_r >> 1
    rpar = pos_r & 1

    g_h, g_t, g_hn, g_tn, g_rel = _gather_rows(
        ent2, rel2,
        epair.reshape(4 * B // CH, CH),
        rpair.reshape(B // CH, CH),
    )

    pars2d = jnp.concatenate([epar, rpar]).reshape(5 * B, 1)

    loss = pl.pallas_call(
        _loss_body,
        grid=(GRID,),
        in_specs=[
            pl.BlockSpec((BLK, 2 * D), lambda i: (i, 0)),
            pl.BlockSpec((BLK, 2 * D), lambda i: (i, 0)),
            pl.BlockSpec((BLK, 2 * D), lambda i: (i, 0)),
            pl.BlockSpec((BLK, 2 * D), lambda i: (i, 0)),
            pl.BlockSpec((BLK, 2 * D), lambda i: (i, 0)),
            pl.BlockSpec((BLK, 1), lambda i: (i, 0)),
            pl.BlockSpec((BLK, 1), lambda i: (i + GRID, 0)),
            pl.BlockSpec((BLK, 1), lambda i: (i + 2 * GRID, 0)),
            pl.BlockSpec((BLK, 1), lambda i: (i + 3 * GRID, 0)),
            pl.BlockSpec((BLK, 1), lambda i: (i + 4 * GRID, 0)),
        ],
        out_specs=pl.BlockSpec((1, 1), lambda i: (0, 0)),
        out_shape=jax.ShapeDtypeStruct((1, 1), jnp.float32),
    )(g_h, g_t, g_hn, g_tn, g_rel, pars2d, pars2d, pars2d, pars2d, pars2d)
    return loss[0, 0]


# speedup vs baseline: 1.9659x; 1.0249x over previous
import dataclasses
import functools

import jax
import jax.numpy as jnp
from jax import lax
from jax.experimental import pallas as pl
from jax.experimental.pallas import tpu as pltpu
from jax.experimental.pallas import tpu_sc as plsc

N_ENT = 1000000
N_REL = 1000
HALF_ENT = N_ENT // 2
HALF_REL = N_REL // 2
D = 64
B = 16384
MARGIN = 1.0

NC = 2
NS = 16
NW = NC * NS

R_PER_W = B // NW
CH = 128
C_PER_SUB = R_PER_W // CH

PACK_BLK = 5000
PACK_GRID = HALF_ENT // PACK_BLK

BLK = 2048
GRID = B // BLK


def _sc_compiler_params():
    cp = pltpu.CompilerParams()
    if "needs_layout_passes" in pltpu.CompilerParams.__dataclass_fields__:
        cp = dataclasses.replace(cp, needs_layout_passes=False)
    return cp


PACK_W = 6400
PACK_NW_FULL = N_ENT // PACK_W
PACK_TAIL = N_ENT - PACK_NW_FULL * PACK_W


def _pack_body(x_hbm, tail_hbm, o_hbm,
               xb0, xb1, zb0, zb1, isem0, isem1, osem0, osem1):
    xbs, zbs = (xb0, xb1), (zb0, zb1)
    isems, osems = (isem0, isem1), (osem0, osem1)
    W, HW, NWF = PACK_W, PACK_W // 2, PACK_NW_FULL

    def start_in(i, b):
        col0 = pl.multiple_of(i * W, 128)
        pltpu.make_async_copy(x_hbm.at[:, pl.ds(col0, W)], xbs[b], isems[b]).start()

    def wait_in(b):
        pltpu.make_async_copy(x_hbm.at[:, pl.ds(0, W)], xbs[b], isems[b]).wait()

    def compute(b):
        y = jnp.transpose(xbs[b][...])
        zbs[b][:, :D] = y[:HW]
        zbs[b][:, D:] = y[HW:]

    def start_out(i, b):
        pltpu.make_async_copy(
            zbs[b], o_hbm.at[pl.ds(pl.multiple_of(i * HW, 8), HW)], osems[b]
        ).start()

    def wait_out(b):
        pltpu.make_async_copy(zbs[b], o_hbm.at[pl.ds(0, HW)], osems[b]).wait()

    start_in(0, 0)

    @pl.loop(0, NWF // 2)
    def _(k):
        i0 = k * 2
        start_in(i0 + 1, 1)
        wait_in(0)

        @pl.when(k > 0)
        def _():
            wait_out(0)

        compute(0)
        start_out(i0, 0)

        @pl.when(k < NWF // 2 - 1)
        def _():
            start_in(i0 + 2, 0)

        wait_in(1)

        @pl.when(k > 0)
        def _():
            wait_out(1)

        compute(1)
        start_out(i0 + 1, 1)

    wait_out(0)
    wait_out(1)

    tcp = pltpu.make_async_copy(
        tail_hbm, o_hbm.at[pl.ds(NWF * W // 2, PACK_TAIL // 2)], osem0)
    tcp.start()
    tcp.wait()


def _pack_entity(ent):
    ent_t = jnp.transpose(ent)
    t = ent[PACK_NW_FULL * PACK_W:]
    tail = jnp.concatenate([t[: PACK_TAIL // 2], t[PACK_TAIL // 2:]], axis=1)
    return pl.pallas_call(
        _pack_body,
        in_specs=[
            pl.BlockSpec(memory_space=pltpu.MemorySpace.HBM),
            pl.BlockSpec(memory_space=pltpu.MemorySpace.HBM),
        ],
        out_specs=pl.BlockSpec(memory_space=pltpu.MemorySpace.HBM),
        out_shape=jax.ShapeDtypeStruct((HALF_ENT, 2 * D), jnp.float32),
        scratch_shapes=[
            pltpu.VMEM((D, PACK_W), jnp.float32),
            pltpu.VMEM((D, PACK_W), jnp.float32),
            pltpu.VMEM((PACK_W // 2, 2 * D), jnp.float32),
            pltpu.VMEM((PACK_W // 2, 2 * D), jnp.float32),
            pltpu.SemaphoreType.DMA,
            pltpu.SemaphoreType.DMA,
            pltpu.SemaphoreType.DMA,
            pltpu.SemaphoreType.DMA,
        ],
    )(ent_t, tail)


def _gather_rows(ent2, rel2, eidx2d, ridx2d):
    mesh = plsc.VectorSubcoreMesh(core_axis_name="core", subcore_axis_name="subcore")

    row_type = jax.ShapeDtypeStruct((B, 2 * D), jnp.float32)

    @functools.partial(
        pl.kernel,
        out_type=(row_type, row_type, row_type, row_type, row_type),
        mesh=mesh,
        scratch_types=[
            pltpu.VMEM((5 * C_PER_SUB, CH), jnp.int32),
            pltpu.VMEM((CH, 2 * D), jnp.float32),
            pltpu.VMEM((CH, 2 * D), jnp.float32),
            pltpu.SemaphoreType.DMA,
            pltpu.SemaphoreType.DMA,
        ],
        compiler_params=_sc_compiler_params(),
    )
    def gk(ent_hbm, rel_hbm, eidx_hbm, ridx_hbm,
           h_out, t_out, hn_out, tn_out, rel_out,
           idx_v, rows_v0, rows_v1, sem0, sem1):
        wid = lax.axis_index("subcore") * NC + lax.axis_index("core")
        rows_vs = (rows_v0, rows_v1)
        sems = (sem0, sem1)

        for k in range(4):
            pltpu.sync_copy(
                eidx_hbm.at[pl.ds(k * (B // CH) + wid * C_PER_SUB, C_PER_SUB)],
                idx_v.at[pl.ds(k * C_PER_SUB, C_PER_SUB)])
        pltpu.sync_copy(
            ridx_hbm.at[pl.ds(wid * C_PER_SUB, C_PER_SUB)],
            idx_v.at[pl.ds(4 * C_PER_SUB, C_PER_SUB)])

        work = []
        for k, out in enumerate((h_out, t_out, hn_out, tn_out)):
            for c in range(C_PER_SUB):
                work.append((ent_hbm, k * C_PER_SUB + c, out, c))
        for c in range(C_PER_SUB):
            work.append((rel_hbm, 4 * C_PER_SUB + c, rel_out, c))

        handles = {}

        def start(n):
            tab, irow, _, _ = work[n]
            handles[n] = pltpu.make_async_copy(
                tab.at[idx_v.at[irow]], rows_vs[n % 2], sems[n % 2])
            handles[n].start()

        def finish(n):
            _, _, out, c = work[n]
            handles.pop(n).wait()
            base = (wid * C_PER_SUB + c) * CH
            pltpu.sync_copy(rows_vs[n % 2], out.at[pl.ds(base, CH)])

        start(0)
        for n in range(len(work)):
            if n + 1 < len(work):
                start(n + 1)
            finish(n)

    return gk(ent2, rel2, eidx2d, ridx2d)


def _loss_body(h_ref, t_ref, hn_ref, tn_ref, r_ref,
               ph_ref, pt_ref, phn_ref, ptn_ref, pr_ref, out_ref):
    i = pl.program_id(0)

    def pick(x2, p_ref):
        p = p_ref[...]
        return jnp.where(p != 0, x2[:, D:], x2[:, :D])

    def nrm(x):
        n = jnp.sqrt(jnp.sum(x * x, axis=1, keepdims=True))
        return x * (1.0 / (n + 1e-12))

    h = nrm(pick(h_ref[...], ph_ref))
    t = nrm(pick(t_ref[...], pt_ref))
    hn = nrm(pick(hn_ref[...], phn_ref))
    tn = nrm(pick(tn_ref[...], ptn_ref))
    r = pick(r_ref[...], pr_ref)
    pos = jnp.sum(jnp.abs(h + r - t), axis=1)
    neg = jnp.sum(jnp.abs(hn + r - tn), axis=1)
    part = jnp.sum(jnp.maximum(MARGIN + pos - neg, 0.0)) * (1.0 / B)

    @pl.when(i == 0)
    def _():
        out_ref[...] = jnp.zeros_like(out_ref)

    out_ref[...] += jnp.reshape(part, (1, 1))


def kernel(entity_embedding, relation_embedding, pos_h, pos_r, pos_t, neg_h, neg_t):
    ent2 = _pack_entity(entity_embedding)
    rel2 = jnp.reshape(relation_embedding, (HALF_REL, 2 * D))

    eidx = jnp.concatenate([pos_h, pos_t, neg_h, neg_t])
    blk = eidx // PACK_W
    rem = eidx % PACK_W
    hw = jnp.where(blk >= PACK_NW_FULL, PACK_TAIL // 2, PACK_W // 2)
    epair = jnp.minimum(blk, PACK_NW_FULL) * (PACK_W // 2) + rem % hw
    epar = rem // hw
    rpair = pos_r >> 1
    rpar = pos_r & 1

    g_h, g_t, g_hn, g_tn, g_rel = _gather_rows(
        ent2, rel2,
        epair.reshape(4 * B // CH, CH),
        rpair.reshape(B // CH, CH),
    )

    pars2d = jnp.concatenate([epar, rpar]).reshape(5 * B, 1)

    loss = pl.pallas_call(
        _loss_body,
        grid=(GRID,),
        in_specs=[
            pl.BlockSpec((BLK, 2 * D), lambda i: (i, 0)),
            pl.BlockSpec((BLK, 2 * D), lambda i: (i, 0)),
            pl.BlockSpec((BLK, 2 * D), lambda i: (i, 0)),
            pl.BlockSpec((BLK, 2 * D), lambda i: (i, 0)),
            pl.BlockSpec((BLK, 2 * D), lambda i: (i, 0)),
            pl.BlockSpec((BLK, 1), lambda i: (i, 0)),
            pl.BlockSpec((BLK, 1), lambda i: (i + GRID, 0)),
            pl.BlockSpec((BLK, 1), lambda i: (i + 2 * GRID, 0)),
            pl.BlockSpec((BLK, 1), lambda i: (i + 3 * GRID, 0)),
            pl.BlockSpec((BLK, 1), lambda i: (i + 4 * GRID, 0)),
        ],
        out_specs=pl.BlockSpec((1, 1), lambda i: (0, 0)),
        out_shape=jax.ShapeDtypeStruct((1, 1), jnp.float32),
    )(g_h, g_t, g_hn, g_tn, g_rel, pars2d, pars2d, pars2d, pars2d, pars2d)
    return loss[0, 0]


# speedup vs baseline: 2.3488x; 1.1948x over previous
import dataclasses
import functools

import jax
import jax.numpy as jnp
from jax import lax
from jax.experimental import pallas as pl
from jax.experimental.pallas import tpu as pltpu
from jax.experimental.pallas import tpu_sc as plsc

N_ENT = 1000000
N_REL = 1000
HALF_ENT = N_ENT // 2
HALF_REL = N_REL // 2
D = 64
B = 16384
MARGIN = 1.0

NC = 2
NS = 16
NW = NC * NS

R_PER_W = B // NW
CH = 128
C_PER_SUB = R_PER_W // CH

PACK_BLK = 5000
PACK_GRID = HALF_ENT // PACK_BLK

BLK = 2048
GRID = B // BLK


def _sc_compiler_params():
    cp = pltpu.CompilerParams()
    if "needs_layout_passes" in pltpu.CompilerParams.__dataclass_fields__:
        cp = dataclasses.replace(cp, needs_layout_passes=False)
    return cp


PACK_W = 25600
PACK_NW_FULL = N_ENT // PACK_W
PACK_TAIL = N_ENT - PACK_NW_FULL * PACK_W


def _pack_body(x_hbm, tail_hbm, o_hbm,
               xb0, xb1, zb0, zb1, isem0, isem1, osem0, osem1):
    xbs, zbs = (xb0, xb1), (zb0, zb1)
    isems, osems = (isem0, isem1), (osem0, osem1)
    W, HW, NWF = PACK_W, PACK_W // 2, PACK_NW_FULL

    def start_in(i, b):
        col0 = pl.multiple_of(i * W, 128)
        pltpu.make_async_copy(x_hbm.at[:, pl.ds(col0, W)], xbs[b], isems[b]).start()

    def wait_in(b):
        pltpu.make_async_copy(x_hbm.at[:, pl.ds(0, W)], xbs[b], isems[b]).wait()

    def compute(b):
        y = jnp.transpose(xbs[b][...])
        zbs[b][:, :D] = y[:HW]
        zbs[b][:, D:] = y[HW:]

    def start_out(i, b):
        pltpu.make_async_copy(
            zbs[b], o_hbm.at[pl.ds(pl.multiple_of(i * HW, 8), HW)], osems[b]
        ).start()

    def wait_out(b):
        pltpu.make_async_copy(zbs[b], o_hbm.at[pl.ds(0, HW)], osems[b]).wait()

    start_in(0, 0)

    @pl.loop(0, NWF // 2)
    def _(k):
        i0 = k * 2
        start_in(i0 + 1, 1)
        wait_in(0)

        @pl.when(k > 0)
        def _():
            wait_out(0)

        compute(0)
        start_out(i0, 0)

        @pl.when(k < NWF // 2 - 1)
        def _():
            start_in(i0 + 2, 0)

        wait_in(1)

        @pl.when(k > 0)
        def _():
            wait_out(1)

        compute(1)
        start_out(i0 + 1, 1)

    wait_out(0)
    wait_out(1)

    tcp = pltpu.make_async_copy(
        tail_hbm, o_hbm.at[pl.ds(NWF * W // 2, PACK_TAIL // 2)], osem0)
    tcp.start()
    tcp.wait()


def _pack_entity(ent):
    ent_t = jnp.transpose(ent)
    t = ent[PACK_NW_FULL * PACK_W:]
    tail = jnp.concatenate([t[: PACK_TAIL // 2], t[PACK_TAIL // 2:]], axis=1)
    return pl.pallas_call(
        _pack_body,
        in_specs=[
            pl.BlockSpec(memory_space=pltpu.MemorySpace.HBM),
            pl.BlockSpec(memory_space=pltpu.MemorySpace.HBM),
        ],
        out_specs=pl.BlockSpec(memory_space=pltpu.MemorySpace.HBM),
        out_shape=jax.ShapeDtypeStruct((HALF_ENT, 2 * D), jnp.float32),
        scratch_shapes=[
            pltpu.VMEM((D, PACK_W), jnp.float32),
            pltpu.VMEM((D, PACK_W), jnp.float32),
            pltpu.VMEM((PACK_W // 2, 2 * D), jnp.float32),
            pltpu.VMEM((PACK_W // 2, 2 * D), jnp.float32),
            pltpu.SemaphoreType.DMA,
            pltpu.SemaphoreType.DMA,
            pltpu.SemaphoreType.DMA,
            pltpu.SemaphoreType.DMA,
        ],
    )(ent_t, tail)


def _gather_rows(ent2, rel2, eidx2d, ridx2d):
    mesh = plsc.VectorSubcoreMesh(core_axis_name="core", subcore_axis_name="subcore")

    row_type = jax.ShapeDtypeStruct((B, 2 * D), jnp.float32)

    @functools.partial(
        pl.kernel,
        out_type=(row_type, row_type, row_type, row_type, row_type),
        mesh=mesh,
        scratch_types=[
            pltpu.VMEM((5 * C_PER_SUB, CH), jnp.int32),
            pltpu.VMEM((CH, 2 * D), jnp.float32),
            pltpu.VMEM((CH, 2 * D), jnp.float32),
            pltpu.SemaphoreType.DMA,
            pltpu.SemaphoreType.DMA,
        ],
        compiler_params=_sc_compiler_params(),
    )
    def gk(ent_hbm, rel_hbm, eidx_hbm, ridx_hbm,
           h_out, t_out, hn_out, tn_out, rel_out,
           idx_v, rows_v0, rows_v1, sem0, sem1):
        wid = lax.axis_index("subcore") * NC + lax.axis_index("core")
        rows_vs = (rows_v0, rows_v1)
        sems = (sem0, sem1)

        for k in range(4):
            pltpu.sync_copy(
                eidx_hbm.at[pl.ds(k * (B // CH) + wid * C_PER_SUB, C_PER_SUB)],
                idx_v.at[pl.ds(k * C_PER_SUB, C_PER_SUB)])
        pltpu.sync_copy(
            ridx_hbm.at[pl.ds(wid * C_PER_SUB, C_PER_SUB)],
            idx_v.at[pl.ds(4 * C_PER_SUB, C_PER_SUB)])

        work = []
        for k, out in enumerate((h_out, t_out, hn_out, tn_out)):
            for c in range(C_PER_SUB):
                work.append((ent_hbm, k * C_PER_SUB + c, out, c))
        for c in range(C_PER_SUB):
            work.append((rel_hbm, 4 * C_PER_SUB + c, rel_out, c))

        handles = {}

        def start(n):
            tab, irow, _, _ = work[n]
            handles[n] = pltpu.make_async_copy(
                tab.at[idx_v.at[irow]], rows_vs[n % 2], sems[n % 2])
            handles[n].start()

        def finish(n):
            _, _, out, c = work[n]
            handles.pop(n).wait()
            base = (wid * C_PER_SUB + c) * CH
            pltpu.sync_copy(rows_vs[n % 2], out.at[pl.ds(base, CH)])

        start(0)
        for n in range(len(work)):
            if n + 1 < len(work):
                start(n + 1)
            finish(n)

    return gk(ent2, rel2, eidx2d, ridx2d)


def _loss_body(h_ref, t_ref, hn_ref, tn_ref, r_ref, p_ref, out_ref):
    i = pl.program_id(0)
    pbits = p_ref[...]

    def pick(x2, bit):
        p = (pbits >> bit) & 1
        return jnp.where(p != 0, x2[:, D:], x2[:, :D])

    def nrm(x):
        n = jnp.sqrt(jnp.sum(x * x, axis=1, keepdims=True))
        return x * (1.0 / (n + 1e-12))

    h = nrm(pick(h_ref[...], 0))
    t = nrm(pick(t_ref[...], 1))
    hn = nrm(pick(hn_ref[...], 2))
    tn = nrm(pick(tn_ref[...], 3))
    r = pick(r_ref[...], 4)
    pos = jnp.sum(jnp.abs(h + r - t), axis=1)
    neg = jnp.sum(jnp.abs(hn + r - tn), axis=1)
    part = jnp.sum(jnp.maximum(MARGIN + pos - neg, 0.0)) * (1.0 / B)

    @pl.when(i == 0)
    def _():
        out_ref[...] = jnp.zeros_like(out_ref)

    out_ref[...] += jnp.reshape(part, (1, 1))


def kernel(entity_embedding, relation_embedding, pos_h, pos_r, pos_t, neg_h, neg_t):
    ent2 = _pack_entity(entity_embedding)
    rel2 = jnp.reshape(relation_embedding, (HALF_REL, 2 * D))

    eidx = jnp.concatenate([pos_h, pos_t, neg_h, neg_t])
    blk = eidx // PACK_W
    rem = eidx % PACK_W
    hw = jnp.where(blk >= PACK_NW_FULL, PACK_TAIL // 2, PACK_W // 2)
    epair = jnp.minimum(blk, PACK_NW_FULL) * (PACK_W // 2) + rem % hw
    epar = rem // hw
    rpair = pos_r >> 1
    rpar = pos_r & 1

    g_h, g_t, g_hn, g_tn, g_rel = _gather_rows(
        ent2, rel2,
        epair.reshape(4 * B // CH, CH),
        rpair.reshape(B // CH, CH),
    )

    ep = epar.reshape(4, B)
    pbits = (ep[0] | (ep[1] << 1) | (ep[2] << 2) | (ep[3] << 3)
             | (rpar << 4)).reshape(B, 1)

    loss = pl.pallas_call(
        _loss_body,
        grid=(GRID,),
        in_specs=[
            pl.BlockSpec((BLK, 2 * D), lambda i: (i, 0)),
            pl.BlockSpec((BLK, 2 * D), lambda i: (i, 0)),
            pl.BlockSpec((BLK, 2 * D), lambda i: (i, 0)),
            pl.BlockSpec((BLK, 2 * D), lambda i: (i, 0)),
            pl.BlockSpec((BLK, 2 * D), lambda i: (i, 0)),
            pl.BlockSpec((BLK, 1), lambda i: (i, 0)),
        ],
        out_specs=pl.BlockSpec((1, 1), lambda i: (0, 0)),
        out_shape=jax.ShapeDtypeStruct((1, 1), jnp.float32),
    )(g_h, g_t, g_hn, g_tn, g_rel, pbits)
    return loss[0, 0]
